# baseline (speedup 1.0000x reference)
import jax
import jax.numpy as jnp
from jax import lax
from jax.experimental import pallas as pl
from jax.experimental.pallas import tpu as pltpu
from jax.experimental.pallas import tpu_sc as plsc

_M = 1_000_000
_D = 15
_B = 16_384
_NB = 2 * _B
_NC = 2
_NS = 16
_NW = _NC * _NS
_NG = _M // 8
_MAIN = 31248
_GROWS = 264
_SCAN_CHUNK = 4096
_NSCAN = _NB // _SCAN_CHUNK
_WS = 8


def _body(mem, idx, updf, out, *scr):
    idxb, gpk = scr[0], scr[1]
    gb = scr[2:2 + _WS]
    pb = scr[2 + _WS:2 + 2 * _WS]
    s_cp = scr[2 + 2 * _WS]
    si = scr[3 + 2 * _WS:3 + 3 * _WS]
    sp = scr[3 + 3 * _WS:3 + 4 * _WS]
    so = scr[3 + 4 * _WS:3 + 5 * _WS]

    wid = lax.axis_index("c") * _NS + lax.axis_index("s")
    lo = ((wid * _NG) // _NW) * 8
    hi = (((wid + 1) * _NG) // _NW) * 8
    rpw = hi - lo
    iot = lax.iota(jnp.int32, 16)

    cp_main = pltpu.make_async_copy(
        mem.at[pl.ds(lo, _MAIN)], out.at[pl.ds(lo, _MAIN)], s_cp)
    cp_main.start()
    has_tail = rpw > _MAIN

    def cp_tail():
        return pltpu.make_async_copy(
            mem.at[pl.ds(lo + _MAIN, 8)], out.at[pl.ds(lo + _MAIN, 8)], s_cp)

    @pl.when(has_tail)
    def _():
        cp_tail().start()

    def scan_chunk(c, cnt_v):
        pltpu.sync_copy(idx.at[pl.ds(c * _SCAN_CHUNK, _SCAN_CHUNK)], idxb)

        def it(i, cnt_v):
            v = idxb[pl.ds(i * 16, 16)]
            m = (v >= lo) & (v < hi)
            pc_v = plsc.all_reduce_population_count(m)
            inc = plsc.cumsum(m.astype(jnp.int32))
            p = cnt_v + inc - 1
            posv = (c * _SCAN_CHUNK) + i * 16 + iot
            e = ((v - lo) << 16) | posv
            plsc.store_scatter(gpk, [p >> 7, p & 127], e, mask=m)
            return cnt_v + pc_v

        return lax.fori_loop(0, _SCAN_CHUNK // 16, it, cnt_v)

    cnt_v = jnp.zeros((16,), jnp.int32)
    for c in range(_NSCAN):
        cnt_v = scan_chunk(c, cnt_v)
    cnt = cnt_v[0]

    cp_main.wait()

    @pl.when(has_tail)
    def _():
        cp_tail().wait()

    def gslice(grp):
        return out.at[pl.ds(grp * 8, 8)]

    def wave(ev, base):
        es = [ev[s] for s in range(_WS)]
        row = [lo + (es[s] >> 16) for s in range(_WS)]
        grp = [row[s] >> 3 for s in range(_WS)]
        pos = [es[s] & 0xFFFF for s in range(_WS)]
        valid = [(base + s) < cnt for s in range(_WS)]
        rem0 = jnp.int32(0)
        for s in range(_WS):
            rem0 = rem0 | jnp.where(valid[s], jnp.int32(1 << s), 0)

        def subwave(remmask):
            rem = [((remmask >> s) & 1) == 1 for s in range(_WS)]
            take = []
            for s in range(_WS):
                conflict = False
                for t in range(s):
                    conflict = conflict | (rem[t] & (grp[t] == grp[s]))
                take.append(rem[s] & jnp.logical_not(conflict))
            for s in range(_WS):
                @pl.when(take[s])
                def _(s=s):
                    pltpu.make_async_copy(gslice(grp[s]), gb[s], si[s]).start()
                    pltpu.make_async_copy(
                        updf.at[pl.ds(pos[s] * 16, 16)], pb[s], sp[s]).start()
            for s in range(_WS):
                @pl.when(take[s])
                def _(s=s):
                    pltpu.make_async_copy(gslice(grp[s]), gb[s], si[s]).wait()
                    pltpu.make_async_copy(
                        updf.at[pl.ds(pos[s] * 16, 16)], pb[s], sp[s]).wait()
                    pvec = pb[s][pl.ds(0, 16)]
                    plsc.store_scatter(
                        gb[s], [jnp.broadcast_to(row[s] & 7, (16,)), iot],
                        pvec, mask=iot < _D)
                    pltpu.make_async_copy(gb[s], gslice(grp[s]), so[s]).start()
            for s in range(_WS):
                @pl.when(take[s])
                def _(s=s):
                    pltpu.make_async_copy(gb[s], gslice(grp[s]), so[s]).wait()
            tkm = jnp.int32(0)
            for s in range(_WS):
                tkm = tkm | jnp.where(take[s], jnp.int32(1 << s), 0)
            return remmask & ~tkm

        lax.while_loop(lambda m: m != 0, subwave, rem0)

    def kv_body(k, z):
        ev = gpk[k >> 3, pl.ds((k & 7) * 16, 16)]
        wave(ev[:8], k * 16)
        wave(ev[8:16], k * 16 + 8)
        return z

    lax.fori_loop(0, (cnt + 15) >> 4, kv_body, 0)


@jax.jit
def _run(mem, idx, updf):
    scratch = [
        pltpu.VMEM((_SCAN_CHUNK,), jnp.int32),
        pltpu.VMEM((_GROWS, 128), jnp.int32),
    ]
    scratch += [pltpu.VMEM((8, _D), jnp.float32) for _ in range(_WS)]
    scratch += [pltpu.VMEM((16,), jnp.float32) for _ in range(_WS)]
    scratch += [pltpu.SemaphoreType.DMA for _ in range(1 + 3 * _WS)]
    f = pl.kernel(
        _body,
        out_type=jax.ShapeDtypeStruct((_M, _D), jnp.float32),
        mesh=plsc.VectorSubcoreMesh(
            core_axis_name="c", subcore_axis_name="s",
            num_cores=_NC, num_subcores=_NS),
        compiler_params=pltpu.CompilerParams(
            needs_layout_passes=False, use_tc_tiling_on_sc=True),
        scratch_types=scratch,
    )
    return f(mem, idx, updf)


def kernel(mem, srcID, src_feature, dstID, dst_feature):
    idx = jnp.concatenate([srcID, dstID], axis=0)
    updf = jnp.pad(jnp.concatenate([src_feature, dst_feature], axis=0),
                   ((0, 0), (0, 1))).reshape(_NB * 16)
    return _run(mem, idx, updf)


# speedup vs baseline: 13.4851x; 13.4851x over previous
import jax
import jax.numpy as jnp
from jax import lax
from jax.experimental import pallas as pl
from jax.experimental.pallas import tpu as pltpu
from jax.experimental.pallas import tpu_sc as plsc

_M = 1_000_000
_D = 15
_B = 16_384
_NB = 2 * _B
_NC = 2
_NS = 16
_NW = _NC * _NS
_NG = _M // 8
_CR = 248
_NCOPY = 126
_GROWS = 264
_SCAN_CHUNK = 4096
_NSCAN = _NB // _SCAN_CHUNK
_WS = 8


def _body(mem, idx, updf, out, *scr):
    idxb, gpk = scr[0], scr[1]
    cb0, cb1 = scr[2], scr[3]
    gb = scr[4:4 + _WS]
    pb = scr[4 + _WS:4 + 2 * _WS]
    s_i0, s_i1, s_o0, s_o1, s_t = scr[4 + 2 * _WS:9 + 2 * _WS]
    si = scr[9 + 2 * _WS:9 + 3 * _WS]
    sp = scr[9 + 3 * _WS:9 + 4 * _WS]
    so = scr[9 + 4 * _WS:9 + 5 * _WS]

    wid = lax.axis_index("c") * _NS + lax.axis_index("s")
    lo = ((wid * _NG) // _NW) * 8
    hi = (((wid + 1) * _NG) // _NW) * 8
    rpw = hi - lo
    iot = lax.iota(jnp.int32, 16)

    bufs = (cb0, cb1)
    sin = (s_i0, s_i1)
    sout = (s_o0, s_o1)

    def cp_in(ci, b):
        return pltpu.make_async_copy(
            mem.at[pl.ds(lo + ci * _CR, _CR)], bufs[b], sin[b])

    def cp_out(ci, b):
        return pltpu.make_async_copy(
            bufs[b], out.at[pl.ds(lo + ci * _CR, _CR)], sout[b])

    cp_in(0, 0).start()
    has_tail = rpw > _NCOPY * _CR

    def cp_tail():
        base = lo + _NCOPY * _CR
        return pltpu.make_async_copy(
            mem.at[pl.ds(base, 8)], out.at[pl.ds(base, 8)], s_t)

    @pl.when(has_tail)
    def _():
        cp_tail().start()

    def scan_chunk(c, cnt_v):
        pltpu.sync_copy(idx.at[pl.ds(c * _SCAN_CHUNK, _SCAN_CHUNK)], idxb)

        def it(i, cnt_v):
            v = idxb[pl.ds(i * 16, 16)]
            m = (v >= lo) & (v < hi)
            pc_v = plsc.all_reduce_population_count(m)
            inc = plsc.cumsum(m.astype(jnp.int32))
            p = cnt_v + inc - 1
            posv = (c * _SCAN_CHUNK) + i * 16 + iot
            e = ((v - lo) << 16) | posv
            plsc.store_scatter(gpk, [p >> 7, p & 127], e, mask=m)
            return cnt_v + pc_v

        return lax.fori_loop(0, _SCAN_CHUNK // 16, it, cnt_v)

    cnt_v = jnp.zeros((16,), jnp.int32)
    for c in range(_NSCAN):
        cnt_v = scan_chunk(c, cnt_v)
    cnt = cnt_v[0]

    def pair(j, z):
        a = 2 * j
        cp_in(a, 0).wait()

        @pl.when(j > 0)
        def _():
            cp_out(a - 1, 1).wait()
        cp_in(a + 1, 1).start()
        cp_out(a, 0).start()

        b = a + 1
        cp_in(b, 1).wait()
        cp_out(b - 1, 0).wait()

        @pl.when(j < _NCOPY // 2 - 1)
        def _():
            cp_in(b + 1, 0).start()
        cp_out(b, 1).start()
        return z

    lax.fori_loop(0, _NCOPY // 2, pair, 0)
    cp_out(_NCOPY - 1, 1).wait()

    @pl.when(has_tail)
    def _():
        cp_tail().wait()

    def gslice(grp):
        return out.at[pl.ds(grp * 8, 8)]

    def wave(ev, base):
        es = [ev[s] for s in range(_WS)]
        row = [lo + (es[s] >> 16) for s in range(_WS)]
        grp = [row[s] >> 3 for s in range(_WS)]
        pos = [es[s] & 0xFFFF for s in range(_WS)]
        valid = [(base + s) < cnt for s in range(_WS)]
        rem0 = jnp.int32(0)
        for s in range(_WS):
            rem0 = rem0 | jnp.where(valid[s], jnp.int32(1 << s), 0)

        def subwave(remmask):
            rem = [((remmask >> s) & 1) == 1 for s in range(_WS)]
            take = []
            for s in range(_WS):
                conflict = False
                for t in range(s):
                    conflict = conflict | (rem[t] & (grp[t] == grp[s]))
                take.append(rem[s] & jnp.logical_not(conflict))
            for s in range(_WS):
                @pl.when(take[s])
                def _(s=s):
                    pltpu.make_async_copy(gslice(grp[s]), gb[s], si[s]).start()
                    pltpu.make_async_copy(
                        updf.at[pl.ds(pos[s] * 16, 16)], pb[s], sp[s]).start()
            for s in range(_WS):
                @pl.when(take[s])
                def _(s=s):
                    pltpu.make_async_copy(gslice(grp[s]), gb[s], si[s]).wait()
                    pltpu.make_async_copy(
                        updf.at[pl.ds(pos[s] * 16, 16)], pb[s], sp[s]).wait()
                    pvec = pb[s][pl.ds(0, 16)]
                    plsc.store_scatter(
                        gb[s], [jnp.broadcast_to(row[s] & 7, (16,)), iot],
                        pvec, mask=iot < _D)
                    pltpu.make_async_copy(gb[s], gslice(grp[s]), so[s]).start()
            for s in range(_WS):
                @pl.when(take[s])
                def _(s=s):
                    pltpu.make_async_copy(gb[s], gslice(grp[s]), so[s]).wait()
            tkm = jnp.int32(0)
            for s in range(_WS):
                tkm = tkm | jnp.where(take[s], jnp.int32(1 << s), 0)
            return remmask & ~tkm

        lax.while_loop(lambda m: m != 0, subwave, rem0)

    def kv_body(k, z):
        ev = gpk[k >> 3, pl.ds((k & 7) * 16, 16)]
        wave(ev[:8], k * 16)
        wave(ev[8:16], k * 16 + 8)
        return z

    lax.fori_loop(0, (cnt + 15) >> 4, kv_body, 0)


@jax.jit
def _run(mem, idx, updf):
    scratch = [
        pltpu.VMEM((_SCAN_CHUNK,), jnp.int32),
        pltpu.VMEM((_GROWS, 128), jnp.int32),
        pltpu.VMEM((_CR, _D), jnp.float32),
        pltpu.VMEM((_CR, _D), jnp.float32),
    ]
    scratch += [pltpu.VMEM((8, _D), jnp.float32) for _ in range(_WS)]
    scratch += [pltpu.VMEM((16,), jnp.float32) for _ in range(_WS)]
    scratch += [pltpu.SemaphoreType.DMA for _ in range(5 + 3 * _WS)]
    f = pl.kernel(
        _body,
        out_type=jax.ShapeDtypeStruct((_M, _D), jnp.float32),
        mesh=plsc.VectorSubcoreMesh(
            core_axis_name="c", subcore_axis_name="s",
            num_cores=_NC, num_subcores=_NS),
        compiler_params=pltpu.CompilerParams(
            needs_layout_passes=False, use_tc_tiling_on_sc=True),
        scratch_types=scratch,
    )
    return f(mem, idx, updf)


def kernel(mem, srcID, src_feature, dstID, dst_feature):
    idx = jnp.concatenate([srcID, dstID], axis=0)
    updf = jnp.pad(jnp.concatenate([src_feature, dst_feature], axis=0),
                   ((0, 0), (0, 1))).reshape(_NB * 16)
    return _run(mem, idx, updf)
